# Initial kernel scaffold; baseline (speedup 1.0000x reference)
import jax
import jax.numpy as jnp
from jax.experimental import pallas as pl


def kernel(img_norm, text_norm, pair_idx, tau_inv, attr_target, obj_target):
    raise NotImplementedError("write your pallas kernel here")


# speedup vs baseline: 3.8750x; 3.8750x over previous
import functools

import jax
import jax.numpy as jnp
from jax import lax
from jax.experimental import pallas as pl
from jax.experimental.pallas import tpu as pltpu
from jax.experimental.pallas import tpu_sc as plsc

BS, D, K, NUM_CLS = 4096, 512, 32768, 256
W_ATTR, W_OBJ = 1.0, 1.0

NC, NS = 2, 16
NW = NC * NS
ROWS_PER_TILE = K // NW
CHUNK = 128
N_CHUNKS = ROWS_PER_TILE // CHUNK
CW = 128
DW = D + CW
ZROWS = 32

CLS_BLK = 64
IMG_BLK = 512
N_IMG_BLKS = BS // IMG_BLK


def _sc_segment_sums(text_norm, labels_a, labels_o, ones_blk, zrows):
    mesh = plsc.VectorSubcoreMesh(core_axis_name="c", subcore_axis_name="s",
                                  num_cores=NC, num_subcores=NS)

    @functools.partial(
        pl.kernel,
        out_type=(
            jax.ShapeDtypeStruct((NW * NUM_CLS, DW), jnp.float32),
            jax.ShapeDtypeStruct((NW * NUM_CLS, DW), jnp.float32),
        ),
        mesh=mesh,
        scratch_types=[
            pltpu.VMEM((CHUNK, DW), jnp.float32),
            pltpu.VMEM((CHUNK,), jnp.int32),
            pltpu.VMEM((CHUNK,), jnp.int32),
            pltpu.VMEM((ZROWS, DW), jnp.float32),
        ],
    )
    def sc_kernel(text_hbm, la_hbm, lo_hbm, ones_hbm, zrows_hbm,
                  sums_a_hbm, sums_o_hbm, rows_v, la_v, lo_v, zrows_v):
        c = lax.axis_index("c")
        s = lax.axis_index("s")
        wid = c * NS + s
        slab = wid * NUM_CLS
        pltpu.sync_copy(zrows_hbm, zrows_v)
        pltpu.sync_copy(ones_hbm, rows_v.at[:, pl.ds(D, CW)])
        for j in range(NUM_CLS // ZROWS):
            dst = pl.ds(slab + j * ZROWS, ZROWS)
            pltpu.sync_copy(zrows_v, sums_a_hbm.at[dst])
            pltpu.sync_copy(zrows_v, sums_o_hbm.at[dst])
        base = wid * ROWS_PER_TILE
        slab_ds = pl.ds(slab, NUM_CLS)
        for i in range(N_CHUNKS):
            r0 = base + i * CHUNK
            pltpu.sync_copy(text_hbm.at[pl.ds(r0, CHUNK)],
                            rows_v.at[:, pl.ds(0, D)])
            pltpu.sync_copy(la_hbm.at[pl.ds(r0, CHUNK)], la_v)
            pltpu.sync_copy(lo_hbm.at[pl.ds(r0, CHUNK)], lo_v)
            pltpu.sync_copy(rows_v, sums_a_hbm.at[slab_ds].at[la_v], add=True)
            pltpu.sync_copy(rows_v, sums_o_hbm.at[slab_ds].at[lo_v], add=True)

    return sc_kernel(text_norm, labels_a, labels_o, ones_blk, zrows)


def _tc_reduce_body(sums_a_ref, sums_o_ref, tau_ref, mean_a_ref, mean_o_ref):
    tau = tau_ref[0, 0]
    for mean_ref, sums_ref in ((mean_a_ref, sums_a_ref),
                               (mean_o_ref, sums_o_ref)):
        total = jnp.sum(sums_ref[...], axis=0)
        cnt = total[:, D:D + 1]
        mean_ref[...] = total[:, :D] * (tau / jnp.maximum(cnt, 1.0))


def _tc_reduce(sums_a, sums_o, tau):
    return pl.pallas_call(
        _tc_reduce_body,
        grid=(NUM_CLS // CLS_BLK,),
        in_specs=[
            pl.BlockSpec((NW, CLS_BLK, DW), lambda i: (0, i, 0)),
            pl.BlockSpec((NW, CLS_BLK, DW), lambda i: (0, i, 0)),
            pl.BlockSpec((1, 1), lambda i: (0, 0)),
        ],
        out_specs=[
            pl.BlockSpec((CLS_BLK, D), lambda i: (i, 0)),
            pl.BlockSpec((CLS_BLK, D), lambda i: (i, 0)),
        ],
        out_shape=[
            jax.ShapeDtypeStruct((NUM_CLS, D), jnp.float32),
            jax.ShapeDtypeStruct((NUM_CLS, D), jnp.float32),
        ],
    )(sums_a, sums_o, tau)


def _tc_loss_body(mean_a_ref, mean_o_ref, img_ref, ta_ref, to_ref,
                  la_ref, lo_ref):
    i = pl.program_id(0)

    @pl.when(i == 0)
    def _():
        la_ref[...] = jnp.zeros((1, 1), jnp.float32)
        lo_ref[...] = jnp.zeros((1, 1), jnp.float32)

    img = img_ref[...]

    def head(mean_ref, tgt_ref):
        logits = lax.dot_general(img, mean_ref[...], (((1,), (1,)), ((), ())),
                                 preferred_element_type=jnp.float32)
        m = jnp.max(logits, axis=1, keepdims=True)
        lse = jnp.log(jnp.sum(jnp.exp(logits - m), axis=1, keepdims=True)) + m
        tgt = tgt_ref[0]
        onehot = lax.broadcasted_iota(jnp.int32, logits.shape, 1) == tgt
        picked = jnp.sum(jnp.where(onehot, logits, 0.0), axis=1, keepdims=True)
        return jnp.sum(lse - picked).reshape(1, 1)

    la_ref[...] += head(mean_a_ref, ta_ref) * (W_ATTR / BS)
    lo_ref[...] += head(mean_o_ref, to_ref) * (W_OBJ / BS)


def _tc_losses(mean_a, mean_o, img_norm, tgt_a, tgt_o):
    full = lambda shape: pl.BlockSpec(shape, lambda i: (0,) * len(shape))
    return pl.pallas_call(
        _tc_loss_body,
        grid=(N_IMG_BLKS,),
        in_specs=[
            full((NUM_CLS, D)),
            full((NUM_CLS, D)),
            pl.BlockSpec((IMG_BLK, D), lambda i: (i, 0)),
            pl.BlockSpec((1, IMG_BLK, 1), lambda i: (i, 0, 0)),
            pl.BlockSpec((1, IMG_BLK, 1), lambda i: (i, 0, 0)),
        ],
        out_specs=[
            pl.BlockSpec((1, 1), lambda i: (0, 0)),
            pl.BlockSpec((1, 1), lambda i: (0, 0)),
        ],
        out_shape=[
            jax.ShapeDtypeStruct((1, 1), jnp.float32),
            jax.ShapeDtypeStruct((1, 1), jnp.float32),
        ],
    )(mean_a, mean_o, img_norm, tgt_a, tgt_o)


def kernel(img_norm, text_norm, pair_idx, tau_inv, attr_target, obj_target):
    labels_a = jnp.asarray(pair_idx[:, 0], jnp.int32)
    labels_o = jnp.asarray(pair_idx[:, 1], jnp.int32)
    ones_blk = jnp.ones((CHUNK, CW), jnp.float32)
    zrows = jnp.zeros((ZROWS, DW), jnp.float32)
    sums_a, sums_o = _sc_segment_sums(
        text_norm, labels_a, labels_o, ones_blk, zrows)
    mean_a, mean_o = _tc_reduce(
        sums_a.reshape(NW, NUM_CLS, DW), sums_o.reshape(NW, NUM_CLS, DW),
        tau_inv.reshape(1, 1))
    tgt_a = attr_target.reshape(N_IMG_BLKS, IMG_BLK, 1)
    tgt_o = obj_target.reshape(N_IMG_BLKS, IMG_BLK, 1)
    loss_a, loss_o = _tc_losses(mean_a, mean_o, img_norm, tgt_a, tgt_o)
    return (loss_a[0, 0], loss_o[0, 0])


# speedup vs baseline: 4.6417x; 1.1979x over previous
import functools

import jax
import jax.numpy as jnp
from jax import lax
from jax.experimental import pallas as pl
from jax.experimental.pallas import tpu as pltpu
from jax.experimental.pallas import tpu_sc as plsc

BS, D, K, NUM_CLS = 4096, 512, 32768, 256
W_ATTR, W_OBJ = 1.0, 1.0

NC, NS = 2, 16
NW = NC * NS
ROWS_PER_TILE = K // NW
CHUNK = 64
N_CHUNKS = ROWS_PER_TILE // CHUNK
ZROWS = 32

CLS_BLK = 64
LBL_BLK = 2048
N_LBL_BLKS = K // LBL_BLK
IMG_BLK = 512
N_IMG_BLKS = BS // IMG_BLK


def _sc_segment_sums(text_norm, labels_a, labels_o, zrows):
    mesh = plsc.VectorSubcoreMesh(core_axis_name="c", subcore_axis_name="s",
                                  num_cores=NC, num_subcores=NS)

    @functools.partial(
        pl.kernel,
        out_type=(
            jax.ShapeDtypeStruct((NW * NUM_CLS, D), jnp.float32),
            jax.ShapeDtypeStruct((NW * NUM_CLS, D), jnp.float32),
        ),
        mesh=mesh,
        scratch_types=[
            pltpu.VMEM((CHUNK, D), jnp.float32),
            pltpu.VMEM((CHUNK, D), jnp.float32),
            pltpu.VMEM((CHUNK,), jnp.int32),
            pltpu.VMEM((CHUNK,), jnp.int32),
            pltpu.VMEM((CHUNK,), jnp.int32),
            pltpu.VMEM((CHUNK,), jnp.int32),
            pltpu.VMEM((ZROWS, D), jnp.float32),
            pltpu.SemaphoreType.DMA,
            pltpu.SemaphoreType.DMA,
            pltpu.SemaphoreType.DMA,
            pltpu.SemaphoreType.DMA,
        ],
    )
    def sc_kernel(text_hbm, la_hbm, lo_hbm, zrows_hbm,
                  sums_a_hbm, sums_o_hbm,
                  rows0_v, rows1_v, la0_v, la1_v, lo0_v, lo1_v, zrows_v,
                  lsem0, lsem1, asem, osem):
        c = lax.axis_index("c")
        s = lax.axis_index("s")
        wid = c * NS + s
        slab = wid * NUM_CLS
        rows_v = (rows0_v, rows1_v)
        la_v = (la0_v, la1_v)
        lo_v = (lo0_v, lo1_v)
        lsem = (lsem0, lsem1)
        base = wid * ROWS_PER_TILE
        slab_ds = pl.ds(slab, NUM_CLS)

        def start_load(i):
            b = i % 2
            r0 = base + i * CHUNK
            return (
                pltpu.async_copy(text_hbm.at[pl.ds(r0, CHUNK)],
                                 rows_v[b], lsem[b]),
                pltpu.async_copy(la_hbm.at[pl.ds(r0, CHUNK)],
                                 la_v[b], lsem[b]),
                pltpu.async_copy(lo_hbm.at[pl.ds(r0, CHUNK)],
                                 lo_v[b], lsem[b]),
            )

        pending = start_load(0)
        pltpu.sync_copy(zrows_hbm, zrows_v)
        for j in range(NUM_CLS // ZROWS):
            dst = pl.ds(slab + j * ZROWS, ZROWS)
            pltpu.sync_copy(zrows_v, sums_a_hbm.at[dst])
            pltpu.sync_copy(zrows_v, sums_o_hbm.at[dst])
        for i in range(N_CHUNKS):
            b = i % 2
            for d in pending:
                d.wait()
            if i + 1 < N_CHUNKS:
                pending = start_load(i + 1)
            da = pltpu.async_copy(rows_v[b], sums_a_hbm.at[slab_ds].at[la_v[b]],
                                  asem, add=True)
            do = pltpu.async_copy(rows_v[b], sums_o_hbm.at[slab_ds].at[lo_v[b]],
                                  osem, add=True)
            da.wait()
            do.wait()

    return sc_kernel(text_norm, labels_a, labels_o, zrows)


def _tc_counts_body(la_ref, lo_ref, cnt_a_ref, cnt_o_ref):
    i = pl.program_id(0)

    @pl.when(i == 0)
    def _():
        cnt_a_ref[...] = jnp.zeros((NUM_CLS, 1), jnp.float32)
        cnt_o_ref[...] = jnp.zeros((NUM_CLS, 1), jnp.float32)

    for lbl_ref, cnt_ref in ((la_ref, cnt_a_ref), (lo_ref, cnt_o_ref)):
        lbl = lbl_ref[0]
        onehot = lax.broadcasted_iota(
            jnp.int32, (NUM_CLS, LBL_BLK), 0) == lbl
        part = jnp.sum(jnp.where(onehot, 1.0, 0.0), axis=1, keepdims=True)
        cnt_ref[...] += part


def _tc_counts(labels_a, labels_o):
    return pl.pallas_call(
        _tc_counts_body,
        grid=(N_LBL_BLKS,),
        in_specs=[
            pl.BlockSpec((1, 1, LBL_BLK), lambda i: (i, 0, 0)),
            pl.BlockSpec((1, 1, LBL_BLK), lambda i: (i, 0, 0)),
        ],
        out_specs=[
            pl.BlockSpec((NUM_CLS, 1), lambda i: (0, 0)),
            pl.BlockSpec((NUM_CLS, 1), lambda i: (0, 0)),
        ],
        out_shape=[
            jax.ShapeDtypeStruct((NUM_CLS, 1), jnp.float32),
            jax.ShapeDtypeStruct((NUM_CLS, 1), jnp.float32),
        ],
    )(labels_a.reshape(N_LBL_BLKS, 1, LBL_BLK),
      labels_o.reshape(N_LBL_BLKS, 1, LBL_BLK))


def _tc_reduce_body(sums_a_ref, sums_o_ref, cnt_a_ref, cnt_o_ref, tau_ref,
                    mean_a_ref, mean_o_ref):
    tau = tau_ref[0, 0]
    for mean_ref, sums_ref, cnt_ref in (
        (mean_a_ref, sums_a_ref, cnt_a_ref),
        (mean_o_ref, sums_o_ref, cnt_o_ref),
    ):
        total = jnp.sum(sums_ref[...], axis=0)
        cnt = cnt_ref[...]
        mean_ref[...] = total * (tau / jnp.maximum(cnt, 1.0))


def _tc_reduce(sums_a, sums_o, cnt_a, cnt_o, tau):
    return pl.pallas_call(
        _tc_reduce_body,
        grid=(NUM_CLS // CLS_BLK,),
        in_specs=[
            pl.BlockSpec((NW, CLS_BLK, D), lambda i: (0, i, 0)),
            pl.BlockSpec((NW, CLS_BLK, D), lambda i: (0, i, 0)),
            pl.BlockSpec((CLS_BLK, 1), lambda i: (i, 0)),
            pl.BlockSpec((CLS_BLK, 1), lambda i: (i, 0)),
            pl.BlockSpec((1, 1), lambda i: (0, 0)),
        ],
        out_specs=[
            pl.BlockSpec((CLS_BLK, D), lambda i: (i, 0)),
            pl.BlockSpec((CLS_BLK, D), lambda i: (i, 0)),
        ],
        out_shape=[
            jax.ShapeDtypeStruct((NUM_CLS, D), jnp.float32),
            jax.ShapeDtypeStruct((NUM_CLS, D), jnp.float32),
        ],
    )(sums_a, sums_o, cnt_a, cnt_o, tau)


def _tc_loss_body(mean_a_ref, mean_o_ref, img_ref, ta_ref, to_ref,
                  la_ref, lo_ref):
    i = pl.program_id(0)

    @pl.when(i == 0)
    def _():
        la_ref[...] = jnp.zeros((1, 1), jnp.float32)
        lo_ref[...] = jnp.zeros((1, 1), jnp.float32)

    img = img_ref[...]

    def head(mean_ref, tgt_ref):
        logits = lax.dot_general(img, mean_ref[...], (((1,), (1,)), ((), ())),
                                 preferred_element_type=jnp.float32)
        m = jnp.max(logits, axis=1, keepdims=True)
        lse = jnp.log(jnp.sum(jnp.exp(logits - m), axis=1, keepdims=True)) + m
        tgt = tgt_ref[0]
        onehot = lax.broadcasted_iota(jnp.int32, logits.shape, 1) == tgt
        picked = jnp.sum(jnp.where(onehot, logits, 0.0), axis=1, keepdims=True)
        return jnp.sum(lse - picked).reshape(1, 1)

    la_ref[...] += head(mean_a_ref, ta_ref) * (W_ATTR / BS)
    lo_ref[...] += head(mean_o_ref, to_ref) * (W_OBJ / BS)


def _tc_losses(mean_a, mean_o, img_norm, tgt_a, tgt_o):
    full = lambda shape: pl.BlockSpec(shape, lambda i: (0,) * len(shape))
    return pl.pallas_call(
        _tc_loss_body,
        grid=(N_IMG_BLKS,),
        in_specs=[
            full((NUM_CLS, D)),
            full((NUM_CLS, D)),
            pl.BlockSpec((IMG_BLK, D), lambda i: (i, 0)),
            pl.BlockSpec((1, IMG_BLK, 1), lambda i: (i, 0, 0)),
            pl.BlockSpec((1, IMG_BLK, 1), lambda i: (i, 0, 0)),
        ],
        out_specs=[
            pl.BlockSpec((1, 1), lambda i: (0, 0)),
            pl.BlockSpec((1, 1), lambda i: (0, 0)),
        ],
        out_shape=[
            jax.ShapeDtypeStruct((1, 1), jnp.float32),
            jax.ShapeDtypeStruct((1, 1), jnp.float32),
        ],
    )(mean_a, mean_o, img_norm, tgt_a, tgt_o)


def kernel(img_norm, text_norm, pair_idx, tau_inv, attr_target, obj_target):
    labels_a = jnp.asarray(pair_idx[:, 0], jnp.int32)
    labels_o = jnp.asarray(pair_idx[:, 1], jnp.int32)
    zrows = jnp.zeros((ZROWS, D), jnp.float32)
    sums_a, sums_o = _sc_segment_sums(text_norm, labels_a, labels_o, zrows)
    cnt_a, cnt_o = _tc_counts(labels_a, labels_o)
    mean_a, mean_o = _tc_reduce(
        sums_a.reshape(NW, NUM_CLS, D), sums_o.reshape(NW, NUM_CLS, D),
        cnt_a, cnt_o, tau_inv.reshape(1, 1))
    tgt_a = attr_target.reshape(N_IMG_BLKS, IMG_BLK, 1)
    tgt_o = obj_target.reshape(N_IMG_BLKS, IMG_BLK, 1)
    loss_a, loss_o = _tc_losses(mean_a, mean_o, img_norm, tgt_a, tgt_o)
    return (loss_a[0, 0], loss_o[0, 0])
